# baseline (speedup 1.0000x reference)
import jax
import jax.numpy as jnp
from jax.experimental import pallas as pl
from jax.experimental.pallas import tpu as pltpu

_SPARSITY = 0.1
_LANES = 1024
_BR = 16
_SUB_ROWS = 128
_SUB_STEPS = 18
_M_STEPS = 15
_STRIPS = 4


def _bits_f(v):
    return jax.lax.bitcast_convert_type(v, jnp.float32)


def _bits_i(v):
    return jax.lax.bitcast_convert_type(v, jnp.int32)


def _run(k, R, N, tau_ref, h_hbm, o_hbm, cur, oth, enc,
         sin_cur, sin_oth, sout_cur, sout_oth):
    n = pl.program_id(0)
    nch = R // _BR
    sub_rows = min(_SUB_ROWS, R)
    rs = R // _STRIPS

    def copy_in(idx, buf, sem, do):
        for s in range(_STRIPS):
            sl = pl.ds(s * rs, rs)
            cp = pltpu.make_async_copy(h_hbm.at[idx, sl], buf.at[sl], sem)
            cp.start() if do else cp.wait()

    def copy_out(buf, idx, sem, do):
        for s in range(_STRIPS):
            sl = pl.ds(s * rs, rs)
            cp = pltpu.make_async_copy(buf.at[sl], o_hbm.at[idx, sl], sem)
            cp.start() if do else cp.wait()

    @pl.when(n == 0)
    def _():
        copy_in(n, cur, sin_cur, True)

    copy_in(n, cur, sin_cur, False)

    kkv = jnp.full((1, 1), float(k), jnp.float32)
    z0 = jnp.full((1, 1), 0, jnp.int32)
    z1 = jnp.full((1, 1), 0x7F800000, jnp.int32)

    def count1(rows, mid_f):
        def acc_body(i, acc):
            blk = jnp.abs(cur[pl.ds(i * _BR, _BR), :])
            return acc + jnp.where(blk >= mid_f, 1.0, 0.0)
        acc = jax.lax.fori_loop(
            0, rows // _BR, acc_body,
            jnp.zeros((_BR, _LANES), jnp.float32), unroll=7)
        return jnp.sum(acc, axis=(0, 1), keepdims=True)

    def count2(rows, mA_f, mB_f, unroll):
        def acc_body(i, accs):
            aA, aB = accs
            blk = jnp.abs(cur[pl.ds(i * _BR, _BR), :])
            aA = aA + jnp.where(blk >= mA_f, 1.0, 0.0)
            aB = aB + jnp.where(blk >= mB_f, 1.0, 0.0)
            return aA, aB
        z = jnp.zeros((_BR, _LANES), jnp.float32)
        aA, aB = jax.lax.fori_loop(0, rows // _BR, acc_body, (z, z),
                                   unroll=unroll)
        return (jnp.sum(aA, axis=(0, 1), keepdims=True),
                jnp.sum(aB, axis=(0, 1), keepdims=True))

    ks = k * (sub_rows * _LANES) // (R * _LANES)
    slack = 380
    kAv = jnp.full((1, 1), float(ks + slack), jnp.float32)
    kBv = jnp.full((1, 1), float(max(ks - slack, 0)), jnp.float32)

    def sub_step(_, carry):
        loA, hiA, loB, hiB = carry
        mA = loA + ((hiA - loA) >> 1)
        mB = loB + ((hiB - loB) >> 1)
        cA, cB = count2(sub_rows, _bits_f(mA), _bits_f(mB), 4)
        bA = cA >= kAv
        bB = cB >= kBv
        return (jnp.where(bA, mA, loA), jnp.where(bA, hiA, mA),
                jnp.where(bB, mB, loB), jnp.where(bB, hiB, mB))

    loA, _, _, hiB = jax.lax.fori_loop(
        0, _SUB_STEPS, sub_step, (z0, z1, z0, z1))

    c_lo, c_hi = count2(R, _bits_f(loA), _bits_f(hiB), 7)
    lo0 = jnp.where(c_lo >= kkv, loA, z0)
    hi0 = jnp.where(c_hi < kkv, hiB, z1)

    w = hi0 - lo0
    fl = (jax.lax.shift_right_logical(_bits_i(w.astype(jnp.float32)), 23)
          - 127)
    sh = jnp.maximum(fl - 14, 0)
    m_hi0 = (w + (jnp.left_shift(jnp.full((1, 1), 1, jnp.int32), sh) - 1)
             ) >> sh

    def enc_pass(i, c):
        sl = pl.ds(i * _BR, _BR)
        u = _bits_i(jnp.abs(cur[sl, :]))
        d = (u - lo0) >> sh
        enc[sl, :] = jnp.clip(d, -32768, 32767).astype(jnp.int16)
        return c
    jax.lax.fori_loop(0, nch, enc_pass, 0, unroll=7)

    @pl.when(n >= 1)
    def _():
        copy_out(oth, n - 1, sout_oth, False)

    @pl.when(n + 1 < N)
    def _():
        copy_in(n + 1, oth, sin_oth, True)

    def count16(m):
        m16 = m.astype(jnp.int16)
        def acc_body(i, acc):
            blk = enc[pl.ds(i * _BR, _BR), :]
            return acc + jnp.where(blk >= m16,
                                   jnp.int16(1), jnp.int16(0))
        acc = jax.lax.fori_loop(
            0, nch, acc_body,
            jnp.zeros((_BR, _LANES), jnp.int16), unroll=7)
        return jnp.sum(acc.astype(jnp.float32), axis=(0, 1), keepdims=True)

    def m_step(_, carry):
        mlo, mhi = carry
        mid = mlo + ((mhi - mlo) >> 1)
        big = count16(mid) >= kkv
        return jnp.where(big, mid, mlo), jnp.where(big, mhi, mid)

    mlo, mhi = jax.lax.fori_loop(
        0, _M_STEPS, m_step, (z0, m_hi0))
    lo2 = lo0 + jnp.left_shift(mlo, sh)
    hi2 = jnp.minimum(lo0 + jnp.left_shift(mhi, sh), hi0)

    lo_s, hi_s = lo2[0, 0], hi2[0, 0]

    def w_cond(carry):
        lo, hi = carry
        return hi - lo > 1

    def w_body(carry):
        lo, hi = carry
        mid = lo + ((hi - lo) >> 1)
        midv = jnp.full((1, 1), 1, jnp.int32) * mid
        big = count1(R, _bits_f(midv))[0, 0] >= jnp.float32(k)
        return jnp.where(big, mid, lo), jnp.where(big, hi, mid)

    lo_s, _ = jax.lax.while_loop(w_cond, w_body, (lo_s, hi_s))

    cutoff_f = _bits_f(jnp.full((1, 1), 1, jnp.int32) * lo_s)
    tau = tau_ref[0, 0]
    one_minus_tau = 1.0 - tau

    def fin(i, c):
        sl = pl.ds(i * _BR, _BR)
        x = cur[sl, :]
        f = jnp.where(jnp.abs(x) >= cutoff_f, 1.0, one_minus_tau)
        cur[sl, :] = x * f
        return c
    jax.lax.fori_loop(0, nch, fin, 0, unroll=7)

    copy_out(cur, n, sout_cur, True)

    @pl.when(n == N - 1)
    def _():
        copy_out(cur, n, sout_cur, False)


def _body(k, R, N, tau_ref, h_hbm, o_hbm, b0, b1, enc,
          si0, si1, so0, so1):
    n = pl.program_id(0)
    bufs = (b0, b1)
    sins = (si0, si1)
    souts = (so0, so1)
    for r in range(2):
        @pl.when(n % 2 == r)
        def _(r=r):
            _run(k, R, N, tau_ref, h_hbm, o_hbm,
                 bufs[r], bufs[1 - r], enc,
                 sins[r], sins[1 - r], souts[r], souts[1 - r])


def kernel(h, tau):
    N, C, H, W = h.shape
    total = C * H * W
    idx = int(_SPARSITY * C * H * W)
    k = idx + 1
    assert total % (_LANES * _BR) == 0
    R = total // _LANES
    assert R % _STRIPS == 0
    hr = h.reshape(N, R, _LANES)
    tau_arr = jnp.asarray(tau, jnp.float32).reshape(1, 1)

    out = pl.pallas_call(
        lambda *refs: _body(k, R, N, *refs),
        grid=(N,),
        in_specs=[
            pl.BlockSpec((1, 1), lambda n: (0, 0)),
            pl.BlockSpec(memory_space=pl.ANY),
        ],
        out_specs=pl.BlockSpec(memory_space=pl.ANY),
        out_shape=jax.ShapeDtypeStruct((N, R, _LANES), jnp.float32),
        scratch_shapes=[
            pltpu.VMEM((R, _LANES), jnp.float32),
            pltpu.VMEM((R, _LANES), jnp.float32),
            pltpu.VMEM((R, _LANES), jnp.int16),
            pltpu.SemaphoreType.DMA,
            pltpu.SemaphoreType.DMA,
            pltpu.SemaphoreType.DMA,
            pltpu.SemaphoreType.DMA,
        ],
    )(tau_arr, hr)
    return out.reshape(N, C, H, W)


# speedup vs baseline: 1.0216x; 1.0216x over previous
import jax
import jax.numpy as jnp
from jax.experimental import pallas as pl
from jax.experimental.pallas import tpu as pltpu

_SPARSITY = 0.1
_LANES = 1024
_BR = 16
_SUB_ROWS = 128
_SUB_STEPS = 18
_M_STEPS = 15
_STRIPS = 4


def _bits_f(v):
    return jax.lax.bitcast_convert_type(v, jnp.float32)


def _bits_i(v):
    return jax.lax.bitcast_convert_type(v, jnp.int32)


def _run(k, R, N, tau_ref, h_hbm, o_hbm, cur, oth, enc,
         sin_cur, sin_oth, sout_cur, sout_oth):
    n = pl.program_id(0)
    nch = R // _BR
    sub_rows = min(_SUB_ROWS, R)
    rs = R // _STRIPS

    def copy_in(idx, buf, sem, do):
        for s in range(_STRIPS):
            sl = pl.ds(s * rs, rs)
            cp = pltpu.make_async_copy(h_hbm.at[idx, sl], buf.at[sl], sem)
            cp.start() if do else cp.wait()

    def copy_out(buf, idx, sem, do):
        for s in range(_STRIPS):
            sl = pl.ds(s * rs, rs)
            cp = pltpu.make_async_copy(buf.at[sl], o_hbm.at[idx, sl], sem)
            cp.start() if do else cp.wait()

    @pl.when(n == 0)
    def _():
        copy_in(n, cur, sin_cur, True)

    copy_in(n, cur, sin_cur, False)

    kkv = jnp.full((1, 1), float(k), jnp.float32)
    z0 = jnp.full((1, 1), 0, jnp.int32)
    z1 = jnp.full((1, 1), 0x7F800000, jnp.int32)

    def count1(rows, mid_f):
        def acc_body(i, acc):
            blk = jnp.abs(cur[pl.ds(i * _BR, _BR), :])
            return acc + jnp.where(blk >= mid_f, 1.0, 0.0)
        acc = jax.lax.fori_loop(
            0, rows // _BR, acc_body,
            jnp.zeros((_BR, _LANES), jnp.float32), unroll=7)
        return jnp.sum(acc, axis=(0, 1), keepdims=True)

    def count2(rows, mA_f, mB_f, unroll):
        def acc_body(i, accs):
            aA, aB = accs
            blk = jnp.abs(cur[pl.ds(i * _BR, _BR), :])
            aA = aA + jnp.where(blk >= mA_f, 1.0, 0.0)
            aB = aB + jnp.where(blk >= mB_f, 1.0, 0.0)
            return aA, aB
        z = jnp.zeros((_BR, _LANES), jnp.float32)
        aA, aB = jax.lax.fori_loop(0, rows // _BR, acc_body, (z, z),
                                   unroll=unroll)
        return (jnp.sum(aA, axis=(0, 1), keepdims=True),
                jnp.sum(aB, axis=(0, 1), keepdims=True))

    ks = k * (sub_rows * _LANES) // (R * _LANES)
    slack = 380
    kAv = jnp.full((1, 1), float(ks + slack), jnp.float32)
    kBv = jnp.full((1, 1), float(max(ks - slack, 0)), jnp.float32)

    def sub_step(_, carry):
        loA, hiA, loB, hiB = carry
        mA = loA + ((hiA - loA) >> 1)
        mB = loB + ((hiB - loB) >> 1)
        cA, cB = count2(sub_rows, _bits_f(mA), _bits_f(mB), 4)
        bA = cA >= kAv
        bB = cB >= kBv
        return (jnp.where(bA, mA, loA), jnp.where(bA, hiA, mA),
                jnp.where(bB, mB, loB), jnp.where(bB, hiB, mB))

    loA, _, _, hiB = jax.lax.fori_loop(
        0, _SUB_STEPS, sub_step, (z0, z1, z0, z1))

    w = hiB - loA
    fl = (jax.lax.shift_right_logical(_bits_i(w.astype(jnp.float32)), 23)
          - 127)
    sh = jnp.maximum(fl - 14, 0)
    m_hi0 = (w + (jnp.left_shift(jnp.full((1, 1), 1, jnp.int32), sh) - 1)
             ) >> sh
    loA_f = _bits_f(loA)
    hiB_f = _bits_f(hiB)

    def ver_enc_body(i, accs):
        aA, aB = accs
        sl = pl.ds(i * _BR, _BR)
        blk = jnp.abs(cur[sl, :])
        aA = aA + jnp.where(blk >= loA_f, 1.0, 0.0)
        aB = aB + jnp.where(blk >= hiB_f, 1.0, 0.0)
        d = (_bits_i(blk) - loA) >> sh
        enc[sl, :] = jnp.clip(d, -32768, 32767).astype(jnp.int16)
        return aA, aB
    zacc = jnp.zeros((_BR, _LANES), jnp.float32)
    accA, accB = jax.lax.fori_loop(0, nch, ver_enc_body, (zacc, zacc),
                                   unroll=7)
    c_lo = jnp.sum(accA, axis=(0, 1), keepdims=True)
    c_hi = jnp.sum(accB, axis=(0, 1), keepdims=True)
    valid = jnp.logical_and(c_lo >= kkv, c_hi < kkv)
    lo0 = jnp.where(valid, loA, z0)
    hi0 = jnp.where(valid, hiB, z1)

    @pl.when(n >= 1)
    def _():
        copy_out(oth, n - 1, sout_oth, False)

    @pl.when(n + 1 < N)
    def _():
        copy_in(n + 1, oth, sin_oth, True)

    def count16(m):
        m16 = m.astype(jnp.int16)
        def acc_body(i, acc):
            blk = enc[pl.ds(i * _BR, _BR), :]
            return acc + jnp.where(blk >= m16,
                                   jnp.int16(1), jnp.int16(0))
        acc = jax.lax.fori_loop(
            0, nch, acc_body,
            jnp.zeros((_BR, _LANES), jnp.int16), unroll=7)
        return jnp.sum(acc.astype(jnp.float32), axis=(0, 1), keepdims=True)

    def m_step(_, carry):
        mlo, mhi = carry
        mid = mlo + ((mhi - mlo) >> 1)
        big = count16(mid) >= kkv
        return jnp.where(big, mid, mlo), jnp.where(big, mhi, mid)

    mlo, mhi = jax.lax.fori_loop(
        0, _M_STEPS, m_step, (z0, m_hi0))
    lo2 = jnp.where(valid, loA + jnp.left_shift(mlo, sh), lo0)
    hi2 = jnp.where(valid,
                    jnp.minimum(loA + jnp.left_shift(mhi, sh), hiB), hi0)

    lo_s, hi_s = lo2[0, 0], hi2[0, 0]

    def w_cond(carry):
        lo, hi = carry
        return hi - lo > 1

    def w_body(carry):
        lo, hi = carry
        mid = lo + ((hi - lo) >> 1)
        midv = jnp.full((1, 1), 1, jnp.int32) * mid
        big = count1(R, _bits_f(midv))[0, 0] >= jnp.float32(k)
        return jnp.where(big, mid, lo), jnp.where(big, hi, mid)

    lo_s, _ = jax.lax.while_loop(w_cond, w_body, (lo_s, hi_s))

    cutoff_f = _bits_f(jnp.full((1, 1), 1, jnp.int32) * lo_s)
    tau = tau_ref[0, 0]
    one_minus_tau = 1.0 - tau

    def fin(i, c):
        sl = pl.ds(i * _BR, _BR)
        x = cur[sl, :]
        f = jnp.where(jnp.abs(x) >= cutoff_f, 1.0, one_minus_tau)
        cur[sl, :] = x * f
        return c
    jax.lax.fori_loop(0, nch, fin, 0, unroll=7)

    copy_out(cur, n, sout_cur, True)

    @pl.when(n == N - 1)
    def _():
        copy_out(cur, n, sout_cur, False)


def _body(k, R, N, tau_ref, h_hbm, o_hbm, b0, b1, enc,
          si0, si1, so0, so1):
    n = pl.program_id(0)
    bufs = (b0, b1)
    sins = (si0, si1)
    souts = (so0, so1)
    for r in range(2):
        @pl.when(n % 2 == r)
        def _(r=r):
            _run(k, R, N, tau_ref, h_hbm, o_hbm,
                 bufs[r], bufs[1 - r], enc,
                 sins[r], sins[1 - r], souts[r], souts[1 - r])


def kernel(h, tau):
    N, C, H, W = h.shape
    total = C * H * W
    idx = int(_SPARSITY * C * H * W)
    k = idx + 1
    assert total % (_LANES * _BR) == 0
    R = total // _LANES
    assert R % _STRIPS == 0
    hr = h.reshape(N, R, _LANES)
    tau_arr = jnp.asarray(tau, jnp.float32).reshape(1, 1)

    out = pl.pallas_call(
        lambda *refs: _body(k, R, N, *refs),
        grid=(N,),
        in_specs=[
            pl.BlockSpec((1, 1), lambda n: (0, 0)),
            pl.BlockSpec(memory_space=pl.ANY),
        ],
        out_specs=pl.BlockSpec(memory_space=pl.ANY),
        out_shape=jax.ShapeDtypeStruct((N, R, _LANES), jnp.float32),
        scratch_shapes=[
            pltpu.VMEM((R, _LANES), jnp.float32),
            pltpu.VMEM((R, _LANES), jnp.float32),
            pltpu.VMEM((R, _LANES), jnp.int16),
            pltpu.SemaphoreType.DMA,
            pltpu.SemaphoreType.DMA,
            pltpu.SemaphoreType.DMA,
            pltpu.SemaphoreType.DMA,
        ],
    )(tau_arr, hr)
    return out.reshape(N, C, H, W)
